# baseline (speedup 1.0000x reference)
import dataclasses
import functools

import jax
import jax.numpy as jnp
from jax import lax
from jax.experimental import pallas as pl
from jax.experimental.pallas import tpu as pltpu
from jax.experimental.pallas import tpu_sc as plsc

N_NODES = 10000
N_EDGES = 320000
N_HID = 128
N_OUT = 64
N_LAYERS = 2
N_GRAPHS = 64
ATOM_FEATS = 9
ATOM_VOCAB = 119

NC, NS = 2, 16
NW = NC * NS
NP = 10240
NPW = NP // NW
ROWS_PER_TILE = NP // NS
K = 128
CPW = 80
EP = NW * CPW * K
NG = EP // K
CPW0 = 116
CPW1 = 2 * CPW - CPW0
NODE_CHUNK = 64
ENC_CHUNKS = NPW // NODE_CHUNK


def _mesh():
    return plsc.VectorSubcoreMesh(core_axis_name="c", subcore_axis_name="s")


def _sc_params():
    cp = pltpu.CompilerParams()
    if "needs_layout_passes" in pltpu.CompilerParams.__dataclass_fields__:
        cp = dataclasses.replace(cp, needs_layout_passes=False)
    return cp


def _encoder(flat_emb, attr_c):
    @functools.partial(
        pl.kernel,
        mesh=_mesh(),
        out_type=jax.ShapeDtypeStruct((NP, N_HID), jnp.float32),
        scratch_types=[
            pltpu.VMEM((ATOM_FEATS, NODE_CHUNK), jnp.int32),
            pltpu.VMEM((ATOM_FEATS, NODE_CHUNK, N_HID), jnp.float32),
            pltpu.VMEM((NODE_CHUNK, N_HID), jnp.float32),
            pltpu.SemaphoreType.DMA,
        ],
        compiler_params=_sc_params(),
    )
    def enc(emb_hbm, attr_hbm, out_hbm, ibuf, rbuf, acc, sem):
        cid = lax.axis_index("c")
        sid = lax.axis_index("s")
        wid = sid * NC + cid
        gbase = wid * ENC_CHUNKS

        @pl.loop(0, ENC_CHUNKS)
        def _(c):
            pltpu.sync_copy(attr_hbm.at[gbase + c], ibuf)
            for f in range(1, ATOM_FEATS):
                for t in range(NODE_CHUNK // 16):
                    sl = pl.ds(t * 16, 16)
                    ibuf[f, sl] = ibuf[f, sl] + (ATOM_VOCAB * f)
            for f in range(ATOM_FEATS):
                pltpu.async_copy(emb_hbm.at[ibuf.at[f]], rbuf.at[f], sem)
            for f in range(ATOM_FEATS):
                pltpu.make_async_copy(emb_hbm.at[ibuf.at[f]], rbuf.at[f],
                                      sem).wait()

            @pl.loop(0, NODE_CHUNK)
            def _(r):
                for j in range(N_HID // 16):
                    sl = pl.ds(j * 16, 16)
                    s = rbuf[0, r, sl]
                    for f in range(1, ATOM_FEATS):
                        s = s + rbuf[f, r, sl]
                    acc[r, sl] = s

            pltpu.sync_copy(
                acc, out_hbm.at[pl.ds((gbase + c) * NODE_CHUNK, NODE_CHUNK)])

    return enc(flat_emb, attr_c)


def _spmm(nrep, edata):
    @functools.partial(
        pl.kernel,
        mesh=_mesh(),
        out_type=(
            jax.ShapeDtypeStruct((NC, NP, N_HID), jnp.float32),
            jax.ShapeDtypeStruct((NC, NP), jnp.float32),
        ),
        scratch_types=[
            pltpu.VMEM((3, K), jnp.int32),
            pltpu.VMEM((3, K), jnp.int32),
            pltpu.VMEM((K,), jnp.float32),
            pltpu.VMEM((K,), jnp.float32),
            pltpu.VMEM((K, N_HID), jnp.float32),
            pltpu.VMEM((K, N_HID), jnp.float32),
            pltpu.VMEM_SHARED((NP, N_HID), jnp.float32),
            pltpu.VMEM_SHARED((NP,), jnp.float32),
            pltpu.SemaphoreType.DMA,
            pltpu.SemaphoreType.DMA,
            pltpu.SemaphoreType.DMA,
            pltpu.SemaphoreType.DMA,
        ],
        compiler_params=_sc_params(),
    )
    def spmm(nrep_hbm, edata_hbm, wsum_hbm, den_hbm,
             ebuf0, ebuf1, av0, av1, rows0, rows1,
             wsum_sh, den_sh, semg0, semg1, sems0, sems1):
        cid = lax.axis_index("c")
        sid = lax.axis_index("s")
        is0 = cid == 0
        gbase = jnp.where(is0, sid * CPW0, NS * CPW0 + sid * CPW1)
        nch = jnp.where(is0, CPW0, CPW1)

        ebufs = (ebuf0, ebuf1)
        avs = (av0, av1)
        rows = (rows0, rows1)
        semgs = (semg0, semg1)
        semss = (sems0, sems1)

        scope_zero = jax.named_scope("spmm_zero")
        scope_zero.__enter__()
        zero16 = jnp.zeros((16,), jnp.float32)

        @pl.loop(0, K)
        def _(r):
            for j in range(N_HID // 16):
                rows0[r, pl.ds(j * 16, 16)] = zero16

        for j in range(K // 16):
            av0[pl.ds(j * 16, 16)] = zero16

        stripe = sid * ROWS_PER_TILE

        @pl.loop(0, ROWS_PER_TILE // K)
        def _(c):
            pltpu.sync_copy(rows0, wsum_sh.at[pl.ds(stripe + c * K, K)])
            pltpu.sync_copy(av0, den_sh.at[pl.ds(stripe + c * K, K)])

        plsc.subcore_barrier()
        scope_zero.__exit__(None, None, None)

        def load_idx_and_a(b, g):
            @pl.when(is0)
            def _():
                pltpu.sync_copy(edata_hbm.at[0, g], ebufs[b])

            @pl.when(jnp.logical_not(is0))
            def _():
                pltpu.sync_copy(edata_hbm.at[1, g], ebufs[b])

            for j in range(K // 16):
                sl = pl.ds(j * 16, 16)
                avs[b][sl] = plsc.bitcast(ebufs[b][2, sl], jnp.float32)

        def start_gather(b):
            @pl.when(is0)
            def _():
                pltpu.async_copy(nrep_hbm.at[0].at[ebufs[b].at[0]], rows[b],
                                 semgs[b])

            @pl.when(jnp.logical_not(is0))
            def _():
                pltpu.async_copy(nrep_hbm.at[1].at[ebufs[b].at[0]], rows[b],
                                 semgs[b])

        def wait_gather(b):
            pltpu.make_async_copy(nrep_hbm.at[0].at[ebufs[b].at[0]], rows[b],
                                  semgs[b]).wait()

        def start_scatter(b):
            pltpu.async_copy(rows[b], wsum_sh.at[ebufs[b].at[1]], semss[b],
                             add=True)
            pltpu.async_copy(avs[b], den_sh.at[ebufs[b].at[1]], semss[b],
                             add=True)

        def wait_scatter(b):
            pltpu.make_async_copy(rows[b], wsum_sh.at[ebufs[b].at[1]],
                                  semss[b]).wait()
            pltpu.make_async_copy(avs[b], den_sh.at[ebufs[b].at[1]],
                                  semss[b]).wait()

        def scale(b):
            @pl.loop(0, K)
            def _(k):
                vs = plsc.load_gather(avs[b], [jnp.full((16,), k, jnp.int32)])
                for j in range(N_HID // 16):
                    sl = pl.ds(j * 16, 16)
                    rows[b][k, sl] = rows[b][k, sl] * vs

        scope_loop = jax.named_scope("spmm_loop")
        scope_loop.__enter__()
        load_idx_and_a(0, gbase)
        start_gather(0)

        def body(i, carry):
            c = i * 2
            wait_gather(0)

            @pl.when(c >= 2)
            def _():
                wait_scatter(1)

            load_idx_and_a(1, gbase + c + 1)
            start_gather(1)
            scale(0)
            start_scatter(0)

            wait_gather(1)
            wait_scatter(0)

            @pl.when(c + 2 < nch)
            def _():
                load_idx_and_a(0, gbase + c + 2)
                start_gather(0)

            scale(1)
            start_scatter(1)
            return carry

        lax.fori_loop(0, nch // 2, body, 0)

        wait_scatter(1)
        plsc.subcore_barrier()
        scope_loop.__exit__(None, None, None)

        scope_out = jax.named_scope("spmm_out")
        scope_out.__enter__()
        @pl.loop(0, ROWS_PER_TILE // K)
        def _(c):
            off = stripe + c * K
            pltpu.sync_copy(wsum_sh.at[pl.ds(off, K)],
                            wsum_hbm.at[cid, pl.ds(off, K)])
            pltpu.sync_copy(den_sh.at[pl.ds(off, K)],
                            den_hbm.at[cid, pl.ds(off, K)])

        scope_out.__exit__(None, None, None)

    return spmm(nrep, edata)


def _dense_body(w_ref, d_ref, x_ref, W_ref, b_ref, g_ref, bb_ref, o_ref):
    ws = w_ref[0] + w_ref[1]
    den = d_ref[0] + d_ref[1]
    aggr = ws * (1.0 / jnp.maximum(den, 1e-30))
    h = jnp.dot(jnp.maximum(aggr, 0.0), W_ref[...],
                preferred_element_type=jnp.float32) + b_ref[...]
    x = h + x_ref[...]
    mean = jnp.mean(x, axis=1, keepdims=True)
    xc = x - mean
    var = jnp.mean(xc * xc, axis=1, keepdims=True)
    o_ref[...] = xc * lax.rsqrt(var + 1e-5) * g_ref[...] + bb_ref[...]


def _dense(wsum, den, nrep, W, b, g, bb):
    grid = NP // 128
    return pl.pallas_call(
        _dense_body,
        grid=(grid,),
        in_specs=[
            pl.BlockSpec((NC, 128, N_HID), lambda i: (0, i, 0)),
            pl.BlockSpec((NC, 128, 1), lambda i: (0, i, 0)),
            pl.BlockSpec((128, N_HID), lambda i: (i, 0)),
            pl.BlockSpec((N_HID, N_HID), lambda i: (0, 0)),
            pl.BlockSpec((1, N_HID), lambda i: (0, 0)),
            pl.BlockSpec((1, N_HID), lambda i: (0, 0)),
            pl.BlockSpec((1, N_HID), lambda i: (0, 0)),
        ],
        out_specs=pl.BlockSpec((128, N_HID), lambda i: (i, 0)),
        out_shape=jax.ShapeDtypeStruct((NP, N_HID), jnp.float32),
    )(wsum, den, nrep, W, b, g, bb)


def _pool_body(x_ref, b_ref, W_ref, ob_ref, o_ref, acc, cnt):
    i = pl.program_id(0)

    @pl.when(i == 0)
    def _():
        acc[...] = jnp.zeros_like(acc)
        cnt[...] = jnp.zeros_like(cnt)

    gids = lax.broadcasted_iota(jnp.int32, (N_GRAPHS, 128), 0)
    onehot = (gids == b_ref[0]).astype(jnp.float32)
    acc[...] += jnp.dot(onehot, x_ref[...], preferred_element_type=jnp.float32)
    cnt[...] += jnp.sum(onehot, axis=1, keepdims=True)

    @pl.when(i == pl.num_programs(0) - 1)
    def _():
        pooled = acc[...] / jnp.maximum(cnt[...], 1.0)
        o_ref[...] = jnp.dot(pooled, W_ref[...],
                             preferred_element_type=jnp.float32) + ob_ref[...]


def _pool(nrep, batch2d, out_W, out_b):
    grid = NP // 128
    return pl.pallas_call(
        _pool_body,
        grid=(grid,),
        in_specs=[
            pl.BlockSpec((128, N_HID), lambda i: (i, 0)),
            pl.BlockSpec((1, 1, 128), lambda i: (i, 0, 0)),
            pl.BlockSpec((N_HID, N_OUT), lambda i: (0, 0)),
            pl.BlockSpec((1, N_OUT), lambda i: (0, 0)),
        ],
        out_specs=pl.BlockSpec((N_GRAPHS, N_OUT), lambda i: (0, 0)),
        out_shape=jax.ShapeDtypeStruct((N_GRAPHS, N_OUT), jnp.float32),
        scratch_shapes=[
            pltpu.VMEM((N_GRAPHS, N_HID), jnp.float32),
            pltpu.VMEM((N_GRAPHS, 1), jnp.float32),
        ],
    )(nrep, batch2d, out_W, out_b)


def kernel(node_attr, edge_index, batch_idx, adv_atts, atom_emb, a_lin_W,
           a_lin_b, ln_g, ln_b, out_W, out_b):
    attr_c = (jnp.pad(node_attr.astype(jnp.int32).T,
                      ((0, 0), (0, NP - N_NODES)))
              .reshape(ATOM_FEATS, NP // NODE_CHUNK, NODE_CHUNK)
              .transpose(1, 0, 2))
    flat_emb = atom_emb.reshape(ATOM_FEATS * ATOM_VOCAB, N_HID)
    src = jnp.pad(edge_index[0].astype(jnp.int32), (0, EP - N_EDGES))
    dst = jnp.pad(edge_index[1].astype(jnp.int32), (0, EP - N_EDGES))
    a_p = jnp.pad(adv_atts, ((0, 0), (0, EP - N_EDGES)))
    edatas = [
        jnp.stack([src.reshape(NG, K), dst.reshape(NG, K),
                   lax.bitcast_convert_type(a_p[l], jnp.int32).reshape(NG, K)],
                  axis=1)
        for l in range(N_LAYERS)
    ]
    batch2d = jnp.pad(batch_idx.astype(jnp.int32), (0, NP - N_NODES),
                      constant_values=N_GRAPHS).reshape(NP // 128, 1, 128)

    nrep = _encoder(flat_emb, attr_c)
    for l in range(N_LAYERS):
        wsum, den = _spmm(jnp.stack([nrep, nrep]),
                          jnp.stack([edatas[l], edatas[l]]))
        nrep = _dense(wsum, den.reshape(NC, NP, 1), nrep, a_lin_W[l],
                      a_lin_b[l].reshape(1, N_HID), ln_g[l].reshape(1, N_HID),
                      ln_b[l].reshape(1, N_HID))
    return _pool(nrep, batch2d, out_W, out_b.reshape(1, N_OUT))


# speedup vs baseline: 1.7902x; 1.7902x over previous
import dataclasses
import functools

import jax
import jax.numpy as jnp
from jax import lax
from jax.experimental import pallas as pl
from jax.experimental.pallas import tpu as pltpu
from jax.experimental.pallas import tpu_sc as plsc

N_NODES = 10000
N_EDGES = 320000
N_HID = 128
N_OUT = 64
N_LAYERS = 2
N_GRAPHS = 64
ATOM_FEATS = 9
ATOM_VOCAB = 119

NC, NS = 2, 16
NW = NC * NS
NP = 10240
NPW = NP // NW
ROWS_PER_TILE = NP // NS
K = 128
CPW = 80
EP = NW * CPW * K
NG = EP // K
CPW0 = 80
CPW1 = 2 * CPW - CPW0
NODE_CHUNK = 64
ENC_CHUNKS = NPW // NODE_CHUNK


def _mesh():
    return plsc.VectorSubcoreMesh(core_axis_name="c", subcore_axis_name="s")


def _sc_params():
    cp = pltpu.CompilerParams()
    if "needs_layout_passes" in pltpu.CompilerParams.__dataclass_fields__:
        cp = dataclasses.replace(cp, needs_layout_passes=False)
    return cp


def _encoder(flat_emb, attr_c):
    @functools.partial(
        pl.kernel,
        mesh=_mesh(),
        out_type=jax.ShapeDtypeStruct((NP, N_HID), jnp.float32),
        scratch_types=[
            pltpu.VMEM((ATOM_FEATS, NODE_CHUNK), jnp.int32),
            pltpu.VMEM((ATOM_FEATS, NODE_CHUNK, N_HID), jnp.float32),
            pltpu.VMEM((NODE_CHUNK, N_HID), jnp.float32),
            pltpu.SemaphoreType.DMA,
        ],
        compiler_params=_sc_params(),
    )
    def enc(emb_hbm, attr_hbm, out_hbm, ibuf, rbuf, acc, sem):
        cid = lax.axis_index("c")
        sid = lax.axis_index("s")
        wid = sid * NC + cid
        gbase = wid * ENC_CHUNKS

        @pl.loop(0, ENC_CHUNKS)
        def _(c):
            pltpu.sync_copy(attr_hbm.at[gbase + c], ibuf)
            for f in range(1, ATOM_FEATS):
                for t in range(NODE_CHUNK // 16):
                    sl = pl.ds(t * 16, 16)
                    ibuf[f, sl] = ibuf[f, sl] + (ATOM_VOCAB * f)
            for f in range(ATOM_FEATS):
                pltpu.async_copy(emb_hbm.at[ibuf.at[f]], rbuf.at[f], sem)
            for f in range(ATOM_FEATS):
                pltpu.make_async_copy(emb_hbm.at[ibuf.at[f]], rbuf.at[f],
                                      sem).wait()

            @pl.loop(0, NODE_CHUNK)
            def _(r):
                for j in range(N_HID // 16):
                    sl = pl.ds(j * 16, 16)
                    s = rbuf[0, r, sl]
                    for f in range(1, ATOM_FEATS):
                        s = s + rbuf[f, r, sl]
                    acc[r, sl] = s

            pltpu.sync_copy(
                acc, out_hbm.at[pl.ds((gbase + c) * NODE_CHUNK, NODE_CHUNK)])

    return enc(flat_emb, attr_c)


def _spmm(nrep, edata):
    @functools.partial(
        pl.kernel,
        mesh=_mesh(),
        out_type=(
            jax.ShapeDtypeStruct((NC, NP, N_HID), jnp.float32),
            jax.ShapeDtypeStruct((NC, NP), jnp.float32),
        ),
        scratch_types=[
            pltpu.VMEM((3, K), jnp.int32),
            pltpu.VMEM((3, K), jnp.int32),
            pltpu.VMEM((K,), jnp.float32),
            pltpu.VMEM((K,), jnp.float32),
            pltpu.VMEM((K, N_HID), jnp.float32),
            pltpu.VMEM((K, N_HID), jnp.float32),
            pltpu.VMEM_SHARED((NP, N_HID), jnp.float32),
            pltpu.VMEM_SHARED((NP,), jnp.float32),
            pltpu.SemaphoreType.DMA,
            pltpu.SemaphoreType.DMA,
            pltpu.SemaphoreType.DMA,
            pltpu.SemaphoreType.DMA,
        ],
        compiler_params=_sc_params(),
    )
    def spmm(nrep_hbm, edata_hbm, wsum_hbm, den_hbm,
             ebuf0, ebuf1, av0, av1, rows0, rows1,
             wsum_sh, den_sh, semg0, semg1, sems0, sems1):
        cid = lax.axis_index("c")
        sid = lax.axis_index("s")
        is0 = cid == 0
        gbase = jnp.where(is0, sid * CPW0, NS * CPW0 + sid * CPW1)
        nch = jnp.where(is0, CPW0, CPW1)

        ebufs = (ebuf0, ebuf1)
        avs = (av0, av1)
        rows = (rows0, rows1)
        semgs = (semg0, semg1)
        semss = (sems0, sems1)

        scope_zero = jax.named_scope("spmm_zero")
        scope_zero.__enter__()
        zero16 = jnp.zeros((16,), jnp.float32)

        @pl.loop(0, K)
        def _(r):
            for j in range(N_HID // 16):
                rows0[r, pl.ds(j * 16, 16)] = zero16

        for j in range(K // 16):
            av0[pl.ds(j * 16, 16)] = zero16

        stripe = sid * ROWS_PER_TILE

        @pl.loop(0, ROWS_PER_TILE // K)
        def _(c):
            pltpu.sync_copy(rows0, wsum_sh.at[pl.ds(stripe + c * K, K)])
            pltpu.sync_copy(av0, den_sh.at[pl.ds(stripe + c * K, K)])

        plsc.subcore_barrier()
        scope_zero.__exit__(None, None, None)

        def load_idx_and_a(b, g):
            pltpu.sync_copy(edata_hbm.at[g], ebufs[b])
            for j in range(K // 16):
                sl = pl.ds(j * 16, 16)
                avs[b][sl] = plsc.bitcast(ebufs[b][2, sl], jnp.float32)

        def start_gather(b):
            pltpu.async_copy(nrep_hbm.at[ebufs[b].at[0]], rows[b], semgs[b])

        def wait_gather(b):
            pltpu.make_async_copy(nrep_hbm.at[ebufs[b].at[0]], rows[b],
                                  semgs[b]).wait()

        def start_scatter(b):
            pltpu.async_copy(rows[b], wsum_sh.at[ebufs[b].at[1]], semss[b],
                             add=True)
            pltpu.async_copy(avs[b], den_sh.at[ebufs[b].at[1]], semss[b],
                             add=True)

        def wait_scatter(b):
            pltpu.make_async_copy(rows[b], wsum_sh.at[ebufs[b].at[1]],
                                  semss[b]).wait()
            pltpu.make_async_copy(avs[b], den_sh.at[ebufs[b].at[1]],
                                  semss[b]).wait()

        def scale(b):
            @pl.loop(0, K)
            def _(k):
                vs = plsc.load_gather(avs[b], [jnp.full((16,), k, jnp.int32)])
                for j in range(N_HID // 16):
                    sl = pl.ds(j * 16, 16)
                    rows[b][k, sl] = rows[b][k, sl] * vs

        scope_loop = jax.named_scope("spmm_loop")
        scope_loop.__enter__()
        load_idx_and_a(0, gbase)
        start_gather(0)

        def body(i, carry):
            c = i * 2
            wait_gather(0)

            @pl.when(c >= 2)
            def _():
                wait_scatter(1)

            load_idx_and_a(1, gbase + c + 1)
            start_gather(1)
            scale(0)
            start_scatter(0)

            wait_gather(1)
            wait_scatter(0)

            @pl.when(c + 2 < nch)
            def _():
                load_idx_and_a(0, gbase + c + 2)
                start_gather(0)

            scale(1)
            start_scatter(1)
            return carry

        lax.fori_loop(0, nch // 2, body, 0)

        wait_scatter(1)
        plsc.subcore_barrier()
        scope_loop.__exit__(None, None, None)

        scope_out = jax.named_scope("spmm_out")
        scope_out.__enter__()
        @pl.loop(0, ROWS_PER_TILE // K)
        def _(c):
            off = stripe + c * K
            pltpu.sync_copy(wsum_sh.at[pl.ds(off, K)],
                            wsum_hbm.at[cid, pl.ds(off, K)])
            pltpu.sync_copy(den_sh.at[pl.ds(off, K)],
                            den_hbm.at[cid, pl.ds(off, K)])

        scope_out.__exit__(None, None, None)

    return spmm(nrep, edata)


def _dense_body(w_ref, d_ref, x_ref, W_ref, b_ref, g_ref, bb_ref, o_ref):
    ws = w_ref[0] + w_ref[1]
    den = d_ref[0] + d_ref[1]
    aggr = ws * (1.0 / jnp.maximum(den, 1e-30))
    h = jnp.dot(jnp.maximum(aggr, 0.0), W_ref[...],
                preferred_element_type=jnp.float32) + b_ref[...]
    x = h + x_ref[...]
    mean = jnp.mean(x, axis=1, keepdims=True)
    xc = x - mean
    var = jnp.mean(xc * xc, axis=1, keepdims=True)
    o_ref[...] = xc * lax.rsqrt(var + 1e-5) * g_ref[...] + bb_ref[...]


def _dense(wsum, den, nrep, W, b, g, bb):
    grid = NP // 128
    return pl.pallas_call(
        _dense_body,
        grid=(grid,),
        in_specs=[
            pl.BlockSpec((NC, 128, N_HID), lambda i: (0, i, 0)),
            pl.BlockSpec((NC, 128, 1), lambda i: (0, i, 0)),
            pl.BlockSpec((128, N_HID), lambda i: (i, 0)),
            pl.BlockSpec((N_HID, N_HID), lambda i: (0, 0)),
            pl.BlockSpec((1, N_HID), lambda i: (0, 0)),
            pl.BlockSpec((1, N_HID), lambda i: (0, 0)),
            pl.BlockSpec((1, N_HID), lambda i: (0, 0)),
        ],
        out_specs=pl.BlockSpec((128, N_HID), lambda i: (i, 0)),
        out_shape=jax.ShapeDtypeStruct((NP, N_HID), jnp.float32),
    )(wsum, den, nrep, W, b, g, bb)


def _pool_body(x_ref, b_ref, W_ref, ob_ref, o_ref, acc, cnt):
    i = pl.program_id(0)

    @pl.when(i == 0)
    def _():
        acc[...] = jnp.zeros_like(acc)
        cnt[...] = jnp.zeros_like(cnt)

    gids = lax.broadcasted_iota(jnp.int32, (N_GRAPHS, 128), 0)
    onehot = (gids == b_ref[0]).astype(jnp.float32)
    acc[...] += jnp.dot(onehot, x_ref[...], preferred_element_type=jnp.float32)
    cnt[...] += jnp.sum(onehot, axis=1, keepdims=True)

    @pl.when(i == pl.num_programs(0) - 1)
    def _():
        pooled = acc[...] / jnp.maximum(cnt[...], 1.0)
        o_ref[...] = jnp.dot(pooled, W_ref[...],
                             preferred_element_type=jnp.float32) + ob_ref[...]


def _pool(nrep, batch2d, out_W, out_b):
    grid = NP // 128
    return pl.pallas_call(
        _pool_body,
        grid=(grid,),
        in_specs=[
            pl.BlockSpec((128, N_HID), lambda i: (i, 0)),
            pl.BlockSpec((1, 1, 128), lambda i: (i, 0, 0)),
            pl.BlockSpec((N_HID, N_OUT), lambda i: (0, 0)),
            pl.BlockSpec((1, N_OUT), lambda i: (0, 0)),
        ],
        out_specs=pl.BlockSpec((N_GRAPHS, N_OUT), lambda i: (0, 0)),
        out_shape=jax.ShapeDtypeStruct((N_GRAPHS, N_OUT), jnp.float32),
        scratch_shapes=[
            pltpu.VMEM((N_GRAPHS, N_HID), jnp.float32),
            pltpu.VMEM((N_GRAPHS, 1), jnp.float32),
        ],
    )(nrep, batch2d, out_W, out_b)


def kernel(node_attr, edge_index, batch_idx, adv_atts, atom_emb, a_lin_W,
           a_lin_b, ln_g, ln_b, out_W, out_b):
    pad_attr = (jnp.arange(NP - N_NODES, dtype=jnp.int32)[None, :]
                + 7 * jnp.arange(ATOM_FEATS, dtype=jnp.int32)[:, None]) % 100
    attr_c = (jnp.concatenate(
        [node_attr.astype(jnp.int32).T, pad_attr], axis=1)
              .reshape(ATOM_FEATS, NP // NODE_CHUNK, NODE_CHUNK)
              .transpose(1, 0, 2))
    flat_emb = atom_emb.reshape(ATOM_FEATS * ATOM_VOCAB, N_HID)
    pad_idx = jnp.arange(EP - N_EDGES, dtype=jnp.int32) % N_NODES
    src = jnp.concatenate([edge_index[0].astype(jnp.int32), pad_idx])
    dst = jnp.concatenate([edge_index[1].astype(jnp.int32), pad_idx])
    a_p = jnp.pad(adv_atts, ((0, 0), (0, EP - N_EDGES)))
    edatas = [
        jnp.stack([src.reshape(NG, K), dst.reshape(NG, K),
                   lax.bitcast_convert_type(a_p[l], jnp.int32).reshape(NG, K)],
                  axis=1)
        for l in range(N_LAYERS)
    ]
    batch2d = jnp.pad(batch_idx.astype(jnp.int32), (0, NP - N_NODES),
                      constant_values=N_GRAPHS).reshape(NP // 128, 1, 128)

    nrep = _encoder(flat_emb, attr_c)
    for l in range(N_LAYERS):
        wsum, den = _spmm(nrep, edatas[l])
        nrep = _dense(wsum, den.reshape(NC, NP, 1), nrep, a_lin_W[l],
                      a_lin_b[l].reshape(1, N_HID), ln_g[l].reshape(1, N_HID),
                      ln_b[l].reshape(1, N_HID))
    return _pool(nrep, batch2d, out_W, out_b.reshape(1, N_OUT))


# speedup vs baseline: 2.1154x; 1.1817x over previous
import dataclasses
import functools

import jax
import jax.numpy as jnp
from jax import lax
from jax.experimental import pallas as pl
from jax.experimental.pallas import tpu as pltpu
from jax.experimental.pallas import tpu_sc as plsc

N_NODES = 10000
N_EDGES = 320000
N_HID = 128
N_OUT = 64
N_LAYERS = 2
N_GRAPHS = 64
ATOM_FEATS = 9
ATOM_VOCAB = 119

NC, NS = 2, 16
NW = NC * NS
NP = 10240
NPW = NP // NW
ROWS_PER_TILE = NP // NS
K = 128
CPW = 80
EP = NW * CPW * K
NG = EP // K
CPW0 = 80
CPW1 = 2 * CPW - CPW0
NODE_CHUNK = 64
ENC_CHUNKS = NPW // NODE_CHUNK


def _mesh():
    return plsc.VectorSubcoreMesh(core_axis_name="c", subcore_axis_name="s")


def _sc_params():
    cp = pltpu.CompilerParams()
    if "needs_layout_passes" in pltpu.CompilerParams.__dataclass_fields__:
        cp = dataclasses.replace(cp, needs_layout_passes=False)
    return cp


def _encoder(flat_emb, attr_c):
    @functools.partial(
        pl.kernel,
        mesh=_mesh(),
        out_type=jax.ShapeDtypeStruct((NP, N_HID), jnp.float32),
        scratch_types=[
            pltpu.VMEM((ATOM_FEATS, NODE_CHUNK), jnp.int32),
            pltpu.VMEM((ATOM_FEATS, NODE_CHUNK, N_HID), jnp.float32),
            pltpu.VMEM((NODE_CHUNK, N_HID), jnp.float32),
            pltpu.SemaphoreType.DMA,
        ],
        compiler_params=_sc_params(),
    )
    def enc(emb_hbm, attr_hbm, out_hbm, ibuf, rbuf, acc, sem):
        cid = lax.axis_index("c")
        sid = lax.axis_index("s")
        wid = sid * NC + cid
        gbase = wid * ENC_CHUNKS

        @pl.loop(0, ENC_CHUNKS)
        def _(c):
            pltpu.sync_copy(attr_hbm.at[gbase + c], ibuf)
            for f in range(1, ATOM_FEATS):
                for t in range(NODE_CHUNK // 16):
                    sl = pl.ds(t * 16, 16)
                    ibuf[f, sl] = ibuf[f, sl] + (ATOM_VOCAB * f)
            for f in range(ATOM_FEATS):
                pltpu.async_copy(emb_hbm.at[ibuf.at[f]], rbuf.at[f], sem)
            for f in range(ATOM_FEATS):
                pltpu.make_async_copy(emb_hbm.at[ibuf.at[f]], rbuf.at[f],
                                      sem).wait()

            @pl.loop(0, NODE_CHUNK)
            def _(r):
                for j in range(N_HID // 16):
                    sl = pl.ds(j * 16, 16)
                    s = rbuf[0, r, sl]
                    for f in range(1, ATOM_FEATS):
                        s = s + rbuf[f, r, sl]
                    acc[r, sl] = s

            pltpu.sync_copy(
                acc, out_hbm.at[pl.ds((gbase + c) * NODE_CHUNK, NODE_CHUNK)])

    return enc(flat_emb, attr_c)


def _spmm(nrep, edata):
    @functools.partial(
        pl.kernel,
        mesh=_mesh(),
        out_type=(
            jax.ShapeDtypeStruct((NC, NP, N_HID), jnp.float32),
            jax.ShapeDtypeStruct((NC, NP), jnp.float32),
        ),
        scratch_types=[
            pltpu.VMEM((3, K), jnp.int32),
            pltpu.VMEM((3, K), jnp.int32),
            pltpu.VMEM((K,), jnp.float32),
            pltpu.VMEM((K,), jnp.float32),
            pltpu.VMEM((K, N_HID), jnp.float32),
            pltpu.VMEM((K, N_HID), jnp.float32),
            pltpu.VMEM_SHARED((NP, N_HID), jnp.float32),
            pltpu.VMEM_SHARED((NP,), jnp.float32),
            pltpu.SemaphoreType.DMA,
            pltpu.SemaphoreType.DMA,
            pltpu.SemaphoreType.DMA,
            pltpu.SemaphoreType.DMA,
        ],
        compiler_params=_sc_params(),
    )
    def spmm(nrep_hbm, edata_hbm, wsum_hbm, den_hbm,
             ebuf0, ebuf1, av0, av1, rows0, rows1,
             wsum_sh, den_sh, semg0, semg1, sems0, sems1):
        cid = lax.axis_index("c")
        sid = lax.axis_index("s")
        is0 = cid == 0
        gbase = jnp.where(is0, sid * CPW0, NS * CPW0 + sid * CPW1)
        nch = jnp.where(is0, CPW0, CPW1)

        ebufs = (ebuf0, ebuf1)
        avs = (av0, av1)
        rows = (rows0, rows1)
        semgs = (semg0, semg1)
        semss = (sems0, sems1)

        scope_zero = jax.named_scope("spmm_zero")
        scope_zero.__enter__()
        zero16 = jnp.zeros((16,), jnp.float32)

        @pl.loop(0, K)
        def _(r):
            for j in range(N_HID // 16):
                rows0[r, pl.ds(j * 16, 16)] = zero16

        for j in range(K // 16):
            av0[pl.ds(j * 16, 16)] = zero16

        stripe = sid * ROWS_PER_TILE

        @pl.loop(0, ROWS_PER_TILE // K)
        def _(c):
            pltpu.sync_copy(rows0, wsum_sh.at[pl.ds(stripe + c * K, K)])
            pltpu.sync_copy(av0, den_sh.at[pl.ds(stripe + c * K, K)])

        plsc.subcore_barrier()
        scope_zero.__exit__(None, None, None)

        def load_idx_and_a(b, g):
            pltpu.sync_copy(edata_hbm.at[g], ebufs[b])
            for j in range(K // 16):
                sl = pl.ds(j * 16, 16)
                avs[b][sl] = plsc.bitcast(ebufs[b][2, sl], jnp.float32)

        def start_gather(b):
            pltpu.async_copy(nrep_hbm.at[ebufs[b].at[0]], rows[b], semgs[b])

        def wait_gather(b):
            pltpu.make_async_copy(nrep_hbm.at[ebufs[b].at[0]], rows[b],
                                  semgs[b]).wait()

        def start_scatter(b):
            pltpu.async_copy(rows[b], wsum_sh.at[ebufs[b].at[1]], semss[b],
                             add=True)
            pltpu.async_copy(avs[b], den_sh.at[ebufs[b].at[1]], semss[b],
                             add=True)

        def wait_scatter(b):
            pltpu.make_async_copy(rows[b], wsum_sh.at[ebufs[b].at[1]],
                                  semss[b]).wait()
            pltpu.make_async_copy(avs[b], den_sh.at[ebufs[b].at[1]],
                                  semss[b]).wait()

        def scale(b):
            @pl.loop(0, K)
            def _(k):
                vs = plsc.load_gather(avs[b], [jnp.full((16,), k, jnp.int32)])
                for j in range(N_HID // 16):
                    sl = pl.ds(j * 16, 16)
                    rows[b][k, sl] = rows[b][k, sl] * vs

        scope_loop = jax.named_scope("spmm_loop")
        scope_loop.__enter__()
        load_idx_and_a(0, gbase)
        start_gather(0)

        def body(i, carry):
            c = i * 2
            wait_gather(0)

            @pl.when(c >= 2)
            def _():
                wait_scatter(1)

            load_idx_and_a(1, gbase + c + 1)
            start_gather(1)
            scale(0)
            start_scatter(0)

            wait_gather(1)
            wait_scatter(0)

            @pl.when(c + 2 < nch)
            def _():
                load_idx_and_a(0, gbase + c + 2)
                start_gather(0)

            scale(1)
            start_scatter(1)
            return carry

        lax.fori_loop(0, nch // 2, body, 0)

        wait_scatter(1)
        plsc.subcore_barrier()
        scope_loop.__exit__(None, None, None)

        scope_out = jax.named_scope("spmm_out")
        scope_out.__enter__()
        @pl.loop(0, ROWS_PER_TILE // K)
        def _(c):
            off = stripe + c * K
            pltpu.sync_copy(wsum_sh.at[pl.ds(off, K)],
                            wsum_hbm.at[cid, pl.ds(off, K)])
            pltpu.sync_copy(den_sh.at[pl.ds(off, K)],
                            den_hbm.at[cid, pl.ds(off, K)])

        scope_out.__exit__(None, None, None)

    return spmm(nrep, edata)


def _dense_body(w_ref, d_ref, x_ref, W_ref, b_ref, g_ref, bb_ref, o_ref):
    ws = w_ref[0] + w_ref[1]
    den = d_ref[0] + d_ref[1]
    aggr = ws * (1.0 / jnp.maximum(den, 1e-30))
    h = jnp.dot(jnp.maximum(aggr, 0.0), W_ref[...],
                preferred_element_type=jnp.float32) + b_ref[...]
    x = h + x_ref[...]
    mean = jnp.mean(x, axis=1, keepdims=True)
    xc = x - mean
    var = jnp.mean(xc * xc, axis=1, keepdims=True)
    o_ref[...] = xc * lax.rsqrt(var + 1e-5) * g_ref[...] + bb_ref[...]


DB = 512


def _dense(wsum, den, nrep, W, b, g, bb):
    grid = NP // DB
    return pl.pallas_call(
        _dense_body,
        grid=(grid,),
        in_specs=[
            pl.BlockSpec((NC, DB, N_HID), lambda i: (0, i, 0)),
            pl.BlockSpec((NC, DB, 1), lambda i: (0, i, 0)),
            pl.BlockSpec((DB, N_HID), lambda i: (i, 0)),
            pl.BlockSpec((N_HID, N_HID), lambda i: (0, 0)),
            pl.BlockSpec((1, N_HID), lambda i: (0, 0)),
            pl.BlockSpec((1, N_HID), lambda i: (0, 0)),
            pl.BlockSpec((1, N_HID), lambda i: (0, 0)),
        ],
        out_specs=pl.BlockSpec((DB, N_HID), lambda i: (i, 0)),
        out_shape=jax.ShapeDtypeStruct((NP, N_HID), jnp.float32),
    )(wsum, den, nrep, W, b, g, bb)


def _pool_body(x_ref, b_ref, W_ref, ob_ref, o_ref, acc, cnt):
    i = pl.program_id(0)

    @pl.when(i == 0)
    def _():
        acc[...] = jnp.zeros_like(acc)
        cnt[...] = jnp.zeros_like(cnt)

    gids = lax.broadcasted_iota(jnp.int32, (N_GRAPHS, DB), 0)
    onehot = (gids == b_ref[0]).astype(jnp.float32)
    acc[...] += jnp.dot(onehot, x_ref[...], preferred_element_type=jnp.float32)
    cnt[...] += jnp.sum(onehot, axis=1, keepdims=True)

    @pl.when(i == pl.num_programs(0) - 1)
    def _():
        pooled = acc[...] / jnp.maximum(cnt[...], 1.0)
        o_ref[...] = jnp.dot(pooled, W_ref[...],
                             preferred_element_type=jnp.float32) + ob_ref[...]


def _pool(nrep, batch2d, out_W, out_b):
    grid = NP // DB
    return pl.pallas_call(
        _pool_body,
        grid=(grid,),
        in_specs=[
            pl.BlockSpec((DB, N_HID), lambda i: (i, 0)),
            pl.BlockSpec((1, 1, DB), lambda i: (i, 0, 0)),
            pl.BlockSpec((N_HID, N_OUT), lambda i: (0, 0)),
            pl.BlockSpec((1, N_OUT), lambda i: (0, 0)),
        ],
        out_specs=pl.BlockSpec((N_GRAPHS, N_OUT), lambda i: (0, 0)),
        out_shape=jax.ShapeDtypeStruct((N_GRAPHS, N_OUT), jnp.float32),
        scratch_shapes=[
            pltpu.VMEM((N_GRAPHS, N_HID), jnp.float32),
            pltpu.VMEM((N_GRAPHS, 1), jnp.float32),
        ],
    )(nrep, batch2d, out_W, out_b)


def kernel(node_attr, edge_index, batch_idx, adv_atts, atom_emb, a_lin_W,
           a_lin_b, ln_g, ln_b, out_W, out_b):
    pad_attr = (jnp.arange(NP - N_NODES, dtype=jnp.int32)[None, :]
                + 7 * jnp.arange(ATOM_FEATS, dtype=jnp.int32)[:, None]) % 100
    attr_c = (jnp.concatenate(
        [node_attr.astype(jnp.int32).T, pad_attr], axis=1)
              .reshape(ATOM_FEATS, NP // NODE_CHUNK, NODE_CHUNK)
              .transpose(1, 0, 2))
    flat_emb = atom_emb.reshape(ATOM_FEATS * ATOM_VOCAB, N_HID)
    pad_idx = jnp.arange(EP - N_EDGES, dtype=jnp.int32) % N_NODES
    src = jnp.concatenate([edge_index[0].astype(jnp.int32), pad_idx])
    dst = jnp.concatenate([edge_index[1].astype(jnp.int32), pad_idx])
    a_p = jnp.pad(adv_atts, ((0, 0), (0, EP - N_EDGES)))
    edatas = [
        jnp.stack([src.reshape(NG, K), dst.reshape(NG, K),
                   lax.bitcast_convert_type(a_p[l], jnp.int32).reshape(NG, K)],
                  axis=1)
        for l in range(N_LAYERS)
    ]
    batch2d = jnp.pad(batch_idx.astype(jnp.int32), (0, NP - N_NODES),
                      constant_values=N_GRAPHS).reshape(NP // DB, 1, DB)

    nrep = _encoder(flat_emb, attr_c)
    for l in range(N_LAYERS):
        wsum, den = _spmm(nrep, edatas[l])
        nrep = _dense(wsum, den.reshape(NC, NP, 1), nrep, a_lin_W[l],
                      a_lin_b[l].reshape(1, N_HID), ln_g[l].reshape(1, N_HID),
                      ln_b[l].reshape(1, N_HID))
    return _pool(nrep, batch2d, out_W, out_b.reshape(1, N_OUT))
